# baseline (speedup 1.0000x reference)
import functools

import jax
import jax.numpy as jnp
from jax import lax
from jax.experimental import pallas as pl
from jax.experimental.pallas import tpu as pltpu
from jax.experimental.pallas import tpu_sc as plsc

_NUM_SEGMENTS = 2
_EMBED_DIM = 1024
_BATCH = 4
_SEQ = 8192
_TOKENS = _BATCH * _SEQ
_T_SC = 3 * _SEQ
_T_TC = _TOKENS - _T_SC
_NW = 32
_TOK_PER_W = _T_SC // _NW
_DRAIN_ROWS = 16
_TC_BLK = 512

_mesh = plsc.VectorSubcoreMesh(core_axis_name="c", subcore_axis_name="s")


@functools.partial(
    pl.kernel,
    mesh=_mesh,
    out_type=jax.ShapeDtypeStruct((_T_SC, _EMBED_DIM), jnp.float32),
    scratch_types=[
        pltpu.SMEM((_TOK_PER_W,), jnp.int32),
        pltpu.VMEM((_NUM_SEGMENTS, _EMBED_DIM), jnp.float32),
        pltpu.VMEM((_DRAIN_ROWS, _EMBED_DIM), jnp.float32),
        pltpu.VMEM_SHARED((_T_SC,), jnp.int32),
        pltpu.SemaphoreType.DMA,
    ],
)
def _segment_gather(idx_hbm, table_hbm, out_hbm, idx_s, table_v, drain_v,
                    idx_sp, sem):
    sid = lax.axis_index("s")
    wid = sid * 2 + lax.axis_index("c")
    base = wid * _TOK_PER_W
    pltpu.sync_copy(table_hbm, table_v)

    @pl.when(sid == 0)
    def _():
        pltpu.sync_copy(idx_hbm, idx_sp)

    plsc.subcore_barrier()
    pltpu.sync_copy(idx_sp.at[pl.ds(base, _TOK_PER_W)], idx_s)

    def body(t, carry):
        s = idx_s[t]
        pltpu.async_copy(table_v.at[s], out_hbm.at[base + t], sem)
        return carry

    lax.fori_loop(0, _TOK_PER_W, body, 0)

    def dbody(i, carry):
        pltpu.make_async_copy(out_hbm.at[pl.ds(base, _DRAIN_ROWS)], drain_v,
                              sem).wait()
        return carry

    lax.fori_loop(0, _TOK_PER_W // _DRAIN_ROWS, dbody, 0)


def _tc_body(idxf_ref, table_ref, out_ref):
    idxf = idxf_ref[...]
    w0 = table_ref[0:1, :]
    w1 = table_ref[1:2, :]
    out_ref[...] = w0 + idxf * (w1 - w0)


_tc_select = pl.pallas_call(
    _tc_body,
    grid=(_T_TC // _TC_BLK,),
    in_specs=[
        pl.BlockSpec((_TC_BLK, 1), lambda i: (i, 0)),
        pl.BlockSpec((_NUM_SEGMENTS, _EMBED_DIM), lambda i: (0, 0)),
    ],
    out_specs=pl.BlockSpec((_TC_BLK, _EMBED_DIM), lambda i: (i, 0)),
    out_shape=jax.ShapeDtypeStruct((_T_TC, _EMBED_DIM), jnp.float32),
)


def kernel(inputs, segment_embed_weights):
    idx = inputs.astype(jnp.int32).reshape(_TOKENS)
    sc_out = _segment_gather(idx[:_T_SC], segment_embed_weights)
    idxf = idx[_T_SC:].astype(jnp.float32).reshape(_T_TC, 1)
    tc_out = _tc_select(idxf, segment_embed_weights)
    out = jnp.concatenate([sc_out, tc_out], axis=0)
    return (out.reshape(_BATCH, _SEQ, _EMBED_DIM), segment_embed_weights)


# speedup vs baseline: 2.3594x; 2.3594x over previous
import functools

import jax
import jax.numpy as jnp
from jax import lax
from jax.experimental import pallas as pl
from jax.experimental.pallas import tpu as pltpu
from jax.experimental.pallas import tpu_sc as plsc

_NUM_SEGMENTS = 2
_EMBED_DIM = 1024
_BATCH = 4
_SEQ = 8192
_TOKENS = _BATCH * _SEQ
_NW = 32
_TOK_PER_W = _TOKENS // _NW
_DRAIN_ROWS = 16

_mesh = plsc.VectorSubcoreMesh(core_axis_name="c", subcore_axis_name="s")


@functools.partial(
    pl.kernel,
    mesh=_mesh,
    out_type=jax.ShapeDtypeStruct((_TOKENS, _EMBED_DIM), jnp.float32),
    scratch_types=[
        pltpu.SMEM((_TOK_PER_W,), jnp.int32),
        pltpu.VMEM((_NUM_SEGMENTS, _EMBED_DIM), jnp.float32),
        pltpu.VMEM((_DRAIN_ROWS, _EMBED_DIM), jnp.float32),
        pltpu.VMEM_SHARED((_TOKENS,), jnp.int32),
        pltpu.SemaphoreType.DMA,
    ],
)
def _segment_gather(idx_hbm, table_hbm, out_hbm, idx_s, table_v, drain_v,
                    idx_sp, sem):
    sid = lax.axis_index("s")
    wid = sid * 2 + lax.axis_index("c")
    base = wid * _TOK_PER_W
    pltpu.sync_copy(table_hbm, table_v)

    @pl.when(sid == 0)
    def _():
        pltpu.sync_copy(idx_hbm, idx_sp)

    plsc.subcore_barrier()
    pltpu.sync_copy(idx_sp.at[pl.ds(base, _TOK_PER_W)], idx_s)

    def body(t, carry):
        s = idx_s[t]
        pltpu.async_copy(table_v.at[s], out_hbm.at[base + t], sem)
        return carry

    lax.fori_loop(0, _TOK_PER_W, body, 0)

    def dbody(i, carry):
        pltpu.make_async_copy(out_hbm.at[pl.ds(base, _DRAIN_ROWS)], drain_v,
                              sem).wait()
        return carry

    lax.fori_loop(0, _TOK_PER_W // _DRAIN_ROWS, dbody, 0)


def kernel(inputs, segment_embed_weights):
    idx = inputs.astype(jnp.int32).reshape(_TOKENS)
    out = _segment_gather(idx, segment_embed_weights)
    return (out.reshape(_BATCH, _SEQ, _EMBED_DIM), segment_embed_weights)


# speedup vs baseline: 2.3684x; 1.0038x over previous
import functools

import jax
import jax.numpy as jnp
from jax import lax
from jax.experimental import pallas as pl
from jax.experimental.pallas import tpu as pltpu
from jax.experimental.pallas import tpu_sc as plsc

_NUM_SEGMENTS = 2
_EMBED_DIM = 1024
_BATCH = 4
_SEQ = 8192
_TOKENS = _BATCH * _SEQ
_NW = 32
_TOK_PER_W = _TOKENS // _NW
_DRAIN_ROWS = 16

_mesh = plsc.VectorSubcoreMesh(core_axis_name="c", subcore_axis_name="s")


@functools.partial(
    pl.kernel,
    mesh=_mesh,
    out_type=jax.ShapeDtypeStruct((_TOKENS, _EMBED_DIM), jnp.float32),
    scratch_types=[
        pltpu.SMEM((_TOK_PER_W,), jnp.int32),
        pltpu.VMEM((_NUM_SEGMENTS, _EMBED_DIM), jnp.float32),
        pltpu.VMEM((_DRAIN_ROWS, _EMBED_DIM), jnp.float32),
        pltpu.VMEM_SHARED((_BATCH, _SEQ), jnp.int32),
        pltpu.SemaphoreType.DMA,
    ],
)
def _segment_gather(idx_hbm, table_hbm, out_hbm, idx_s, table_v, drain_v,
                    idx_sp, sem):
    sid = lax.axis_index("s")
    wid = sid * 2 + lax.axis_index("c")
    base = wid * _TOK_PER_W
    pltpu.sync_copy(table_hbm, table_v)

    @pl.when(sid == 0)
    def _():
        pltpu.sync_copy(idx_hbm, idx_sp)

    plsc.subcore_barrier()
    _W_PER_B = _SEQ // _TOK_PER_W
    pltpu.sync_copy(
        idx_sp.at[wid // _W_PER_B,
                  pl.ds((wid % _W_PER_B) * _TOK_PER_W, _TOK_PER_W)], idx_s)

    def body(t, carry):
        s = idx_s[t]
        pltpu.async_copy(table_v.at[s], out_hbm.at[base + t], sem)
        return carry

    lax.fori_loop(0, _TOK_PER_W, body, 0)

    def dbody(i, carry):
        pltpu.make_async_copy(out_hbm.at[pl.ds(base, _DRAIN_ROWS)], drain_v,
                              sem).wait()
        return carry

    lax.fori_loop(0, _TOK_PER_W // _DRAIN_ROWS, dbody, 0)


def kernel(inputs, segment_embed_weights):
    idx = inputs.astype(jnp.int32)
    out = _segment_gather(idx, segment_embed_weights)
    return (out.reshape(_BATCH, _SEQ, _EMBED_DIM), segment_embed_weights)
